# baseline (speedup 1.0000x reference)
import functools

import jax
import jax.numpy as jnp
from jax import lax
from jax.experimental import pallas as pl
from jax.experimental.pallas import tpu as pltpu
from jax.experimental.pallas import tpu_sc as plsc

_N = 10000
_H = 128
_E = 320000
_NET = 4
_NSTEPS = 3
_NCLS = 16

_NC = 2
_NS = 16
_NW = _NC * _NS
_K = 128
_NCHUNK = 79
_CHA = 53
_CHB = 105
_CHMAX = 105
_EPW = _NCHUNK * _K
_EPAD = _NW * _EPW
_NACC = 10112
_ZROW = _NACC // _NS
_OROW = _NACC // _NS

_CH = 4
_BN = 1000
_NB = _N // _BN


def _etyp_body(h_ref, wt_ref, b_ref, t_ref):
    t_ref[0] = (
        jnp.dot(h_ref[...], wt_ref[0], preferred_element_type=jnp.float32)
        + b_ref[0]
    )


def _etype_linear(h, wt, b3):
    return pl.pallas_call(
        _etyp_body,
        grid=(_NET, _NB),
        in_specs=[
            pl.BlockSpec((_BN, _H), lambda e, i: (i, 0)),
            pl.BlockSpec((1, _H, _H), lambda e, i: (e, 0, 0)),
            pl.BlockSpec((1, 1, _H), lambda e, i: (e, 0, 0)),
        ],
        out_specs=pl.BlockSpec((1, _BN, _H), lambda e, i: (e, i, 0)),
        out_shape=jax.ShapeDtypeStruct((_NET, _N, _H), jnp.float32),
    )(h, wt, b3)


def _gru_core(ap_ref, h_ref, wih_ref, whh_ref, bih_ref, bhh_ref):
    a = ap_ref[0] + ap_ref[1]
    h = h_ref[...]
    gi = jnp.dot(a, wih_ref[...], preferred_element_type=jnp.float32) + bih_ref[...]
    gh = jnp.dot(h, whh_ref[...], preferred_element_type=jnp.float32) + bhh_ref[...]
    r = jax.nn.sigmoid(gi[:, :_H] + gh[:, :_H])
    z = jax.nn.sigmoid(gi[:, _H:2 * _H] + gh[:, _H:2 * _H])
    n = jnp.tanh(gi[:, 2 * _H:] + r * gh[:, 2 * _H:])
    return (1.0 - z) * n + z * h


def _gru_body(ap_ref, h_ref, wih_ref, whh_ref, bih_ref, bhh_ref, o_ref):
    o_ref[...] = _gru_core(ap_ref, h_ref, wih_ref, whh_ref, bih_ref, bhh_ref)


def _gru(ap, h, wiht, whht, bih2, bhh2):
    return pl.pallas_call(
        _gru_body,
        grid=(_NB,),
        in_specs=[
            pl.BlockSpec((_NC, _BN, _H), lambda i: (0, i, 0)),
            pl.BlockSpec((_BN, _H), lambda i: (i, 0)),
            pl.BlockSpec((_H, 3 * _H), lambda i: (0, 0)),
            pl.BlockSpec((_H, 3 * _H), lambda i: (0, 0)),
            pl.BlockSpec((1, 3 * _H), lambda i: (0, 0)),
            pl.BlockSpec((1, 3 * _H), lambda i: (0, 0)),
        ],
        out_specs=pl.BlockSpec((_BN, _H), lambda i: (i, 0)),
        out_shape=jax.ShapeDtypeStruct((_N, _H), jnp.float32),
    )(ap, h, wiht, whht, bih2, bhh2)


def _gru_fused_body(ap_ref, h_ref, wih_ref, whh_ref, bih_ref, bhh_ref,
                    wt_ref, b_ref, o_ref, t_ref):
    hn = _gru_core(ap_ref, h_ref, wih_ref, whh_ref, bih_ref, bhh_ref)
    o_ref[...] = hn
    for e in range(_NET):
        t_ref[e] = (
            jnp.dot(hn, wt_ref[e], preferred_element_type=jnp.float32)
            + b_ref[e]
        )


def _gru_fused(ap, h, wiht, whht, bih2, bhh2, wt, b3):
    return pl.pallas_call(
        _gru_fused_body,
        grid=(_NB,),
        in_specs=[
            pl.BlockSpec((_NC, _BN, _H), lambda i: (0, i, 0)),
            pl.BlockSpec((_BN, _H), lambda i: (i, 0)),
            pl.BlockSpec((_H, 3 * _H), lambda i: (0, 0)),
            pl.BlockSpec((_H, 3 * _H), lambda i: (0, 0)),
            pl.BlockSpec((1, 3 * _H), lambda i: (0, 0)),
            pl.BlockSpec((1, 3 * _H), lambda i: (0, 0)),
            pl.BlockSpec((_NET, _H, _H), lambda i: (0, 0, 0)),
            pl.BlockSpec((_NET, 1, _H), lambda i: (0, 0, 0)),
        ],
        out_specs=(
            pl.BlockSpec((_BN, _H), lambda i: (i, 0)),
            pl.BlockSpec((_NET, _BN, _H), lambda i: (0, i, 0)),
        ),
        out_shape=(
            jax.ShapeDtypeStruct((_N, _H), jnp.float32),
            jax.ShapeDtypeStruct((_NET, _N, _H), jnp.float32),
        ),
    )(ap, h, wiht, whht, bih2, bhh2, wt, b3)


def _pool_body(h_ref, w_ref, b_ref, o_ref):
    pooled = jnp.mean(h_ref[...], axis=0, keepdims=True)
    o_ref[...] = (
        jnp.dot(pooled, w_ref[...], preferred_element_type=jnp.float32)
        + b_ref[...]
    )


def _pool_fc(h, fcwt, fcb2):
    return pl.pallas_call(
        _pool_body,
        out_shape=jax.ShapeDtypeStruct((1, _NCLS), jnp.float32),
    )(h, fcwt, fcb2)


@functools.cache
def _sc_message_pass_fn():
    mesh = plsc.VectorSubcoreMesh(core_axis_name="c", subcore_axis_name="s")

    @functools.partial(
        pl.kernel,
        mesh=mesh,
        out_type=jax.ShapeDtypeStruct((_NC, _NACC, _H), jnp.float32),
        scratch_types=[
            pltpu.VMEM((_K,), jnp.int32),
            pltpu.VMEM((_K,), jnp.int32),
            pltpu.VMEM((_K,), jnp.int32),
            pltpu.VMEM((_K,), jnp.int32),
            pltpu.VMEM((_K, _H), jnp.float32),
            pltpu.VMEM((_K, _H), jnp.float32),
            pltpu.VMEM_SHARED((_NACC, _H), jnp.float32),
            pltpu.VMEM_SHARED((_NS, _CHMAX * _K), jnp.int32),
            pltpu.SemaphoreType.DMA,
            pltpu.SemaphoreType.DMA,
            pltpu.SemaphoreType.DMA,
            pltpu.SemaphoreType.DMA,
        ],
    )
    def _sc_message_pass(t_hbm, gidx_hbm, didx_hbm, zeros_hbm, out_hbm,
                         gv, dv, gv1, dv1, rows, rows1, acc, idxsh,
                         semg0, semg1, sems0, sems1):
        c = lax.axis_index("c")
        s = lax.axis_index("s")
        wid = s * _NC + c
        pltpu.sync_copy(zeros_hbm.at[pl.ds(s * _ZROW, _ZROW)],
                        acc.at[pl.ds(s * _ZROW, _ZROW)])
        nch = jnp.where(c == 0, _CHA, _CHB)
        base = jnp.where(c == 0, s * _CHA, _NS * _CHA + s * _CHB) * _K
        pltpu.sync_copy(gidx_hbm.at[pl.ds(base, _CHMAX * _K)], idxsh.at[s])
        plsc.subcore_barrier()

        def chunk(j, gvb, dvb, rowsb, semg, sems, first):
            pltpu.sync_copy(idxsh.at[s, pl.ds(j * _K, _K)], gvb)
            if not first:
                pltpu.make_async_copy(rowsb, acc.at[dvb], sems).wait()
            cp = pltpu.async_copy(t_hbm.at[gvb], rowsb, semg)
            pltpu.sync_copy(didx_hbm.at[pl.ds(base + j * _K, _K)], dvb)
            cp.wait()
            pltpu.async_copy(rowsb, acc.at[dvb], sems, add=True)

        chunk(0, gv, dv, rows, semg0, sems0, True)
        chunk(1, gv1, dv1, rows1, semg1, sems1, True)

        def body(p, carry):
            chunk(2 * p, gv, dv, rows, semg0, sems0, False)
            chunk(2 * p + 1, gv1, dv1, rows1, semg1, sems1, False)
            return carry

        lax.fori_loop(1, nch // 2, body, 0)
        chunk(nch - 1, gv, dv, rows, semg0, sems0, False)
        pltpu.make_async_copy(rows, acc.at[dv], sems0).wait()
        pltpu.make_async_copy(rows1, acc.at[dv1], sems1).wait()
        plsc.subcore_barrier()
        pltpu.sync_copy(acc.at[pl.ds(s * _OROW, _OROW)],
                        out_hbm.at[c, pl.ds(s * _OROW, _OROW)])

    return _sc_message_pass


def kernel(node_features, edge_index, etypes, W0, b0, Wih0, Whh0, bih0, bhh0,
           W1, b1, Wih1, Whh1, bih1, bhh1, fcW, fcb):
    src = edge_index[0]
    dst = edge_index[1]
    gidx = etypes * _N + src
    pad = _EPAD - _E
    gidx = jnp.concatenate([gidx, jnp.zeros((pad,), jnp.int32)])
    didx = jnp.concatenate([dst, jnp.full((pad,), _N, jnp.int32)])
    zeros = jnp.zeros((_NACC, _H), jnp.float32)

    h = node_features
    layers = []
    for (W, b, Wih, Whh, bih, bhh) in (
        (W0, b0, Wih0, Whh0, bih0, bhh0),
        (W1, b1, Wih1, Whh1, bih1, bhh1),
    ):
        layers.append((jnp.transpose(W, (0, 2, 1)), b[:, None, :], Wih.T,
                       Whh.T, bih[None, :], bhh[None, :]))

    nsteps = 2 * _NSTEPS
    t = _etype_linear(h, layers[0][0], layers[0][1])
    for k in range(nsteps):
        wt, b3, wiht, whht, bih2, bhh2 = layers[k // _NSTEPS]
        ap = _sc_message_pass_fn()(t.reshape(_NET * _N, _H), gidx, didx, zeros)
        if k < nsteps - 1:
            nwt, nb3 = layers[(k + 1) // _NSTEPS][:2]
            h, t = _gru_fused(ap, h, wiht, whht, bih2, bhh2, nwt, nb3)
        else:
            h = _gru(ap, h, wiht, whht, bih2, bhh2)

    return _pool_fc(h, fcW.T, fcb[None, :])


# speedup vs baseline: 1.1780x; 1.1780x over previous
import functools

import jax
import jax.numpy as jnp
from jax import lax
from jax.experimental import pallas as pl
from jax.experimental.pallas import tpu as pltpu
from jax.experimental.pallas import tpu_sc as plsc

_N = 10000
_H = 128
_E = 320000
_NET = 4
_NSTEPS = 3
_NCLS = 16

_NC = 2
_NS = 16
_NW = _NC * _NS
_K = 128
_NCHUNK = 79
_CHA = 105
_CHB = 53
_CHMAX = 105
_EPW = _NCHUNK * _K
_EPAD = _NW * _EPW
_NACC = 10112
_ZROW = _NACC // _NS
_OROW = _NACC // _NS

_CH = 4
_BN = 1000
_NB = _N // _BN


def _etyp_body(h_ref, wt_ref, b_ref, t_ref):
    t_ref[0] = (
        jnp.dot(h_ref[...], wt_ref[0], preferred_element_type=jnp.float32)
        + b_ref[0]
    )


def _etype_linear(h, wt, b3):
    return pl.pallas_call(
        _etyp_body,
        grid=(_NET, _NB),
        in_specs=[
            pl.BlockSpec((_BN, _H), lambda e, i: (i, 0)),
            pl.BlockSpec((1, _H, _H), lambda e, i: (e, 0, 0)),
            pl.BlockSpec((1, 1, _H), lambda e, i: (e, 0, 0)),
        ],
        out_specs=pl.BlockSpec((1, _BN, _H), lambda e, i: (e, i, 0)),
        out_shape=jax.ShapeDtypeStruct((_NET, _N, _H), jnp.float32),
    )(h, wt, b3)


def _gru_core(ap_ref, h_ref, wih_ref, whh_ref, bih_ref, bhh_ref):
    a = ap_ref[0] + ap_ref[1]
    h = h_ref[...]
    gi = jnp.dot(a, wih_ref[...], preferred_element_type=jnp.float32) + bih_ref[...]
    gh = jnp.dot(h, whh_ref[...], preferred_element_type=jnp.float32) + bhh_ref[...]
    r = jax.nn.sigmoid(gi[:, :_H] + gh[:, :_H])
    z = jax.nn.sigmoid(gi[:, _H:2 * _H] + gh[:, _H:2 * _H])
    n = jnp.tanh(gi[:, 2 * _H:] + r * gh[:, 2 * _H:])
    return (1.0 - z) * n + z * h


def _gru_body(ap_ref, h_ref, wih_ref, whh_ref, bih_ref, bhh_ref, o_ref):
    o_ref[...] = _gru_core(ap_ref, h_ref, wih_ref, whh_ref, bih_ref, bhh_ref)


def _gru(ap, h, wiht, whht, bih2, bhh2):
    return pl.pallas_call(
        _gru_body,
        grid=(_NB,),
        in_specs=[
            pl.BlockSpec((_NC, _BN, _H), lambda i: (0, i, 0)),
            pl.BlockSpec((_BN, _H), lambda i: (i, 0)),
            pl.BlockSpec((_H, 3 * _H), lambda i: (0, 0)),
            pl.BlockSpec((_H, 3 * _H), lambda i: (0, 0)),
            pl.BlockSpec((1, 3 * _H), lambda i: (0, 0)),
            pl.BlockSpec((1, 3 * _H), lambda i: (0, 0)),
        ],
        out_specs=pl.BlockSpec((_BN, _H), lambda i: (i, 0)),
        out_shape=jax.ShapeDtypeStruct((_N, _H), jnp.float32),
    )(ap, h, wiht, whht, bih2, bhh2)


def _gru_fused_body(ap_ref, h_ref, wih_ref, whh_ref, bih_ref, bhh_ref,
                    wt_ref, b_ref, o_ref, t_ref):
    hn = _gru_core(ap_ref, h_ref, wih_ref, whh_ref, bih_ref, bhh_ref)
    o_ref[...] = hn
    for e in range(_NET):
        t_ref[e] = (
            jnp.dot(hn, wt_ref[e], preferred_element_type=jnp.float32)
            + b_ref[e]
        )


def _gru_fused(ap, h, wiht, whht, bih2, bhh2, wt, b3):
    return pl.pallas_call(
        _gru_fused_body,
        grid=(_NB,),
        in_specs=[
            pl.BlockSpec((_NC, _BN, _H), lambda i: (0, i, 0)),
            pl.BlockSpec((_BN, _H), lambda i: (i, 0)),
            pl.BlockSpec((_H, 3 * _H), lambda i: (0, 0)),
            pl.BlockSpec((_H, 3 * _H), lambda i: (0, 0)),
            pl.BlockSpec((1, 3 * _H), lambda i: (0, 0)),
            pl.BlockSpec((1, 3 * _H), lambda i: (0, 0)),
            pl.BlockSpec((_NET, _H, _H), lambda i: (0, 0, 0)),
            pl.BlockSpec((_NET, 1, _H), lambda i: (0, 0, 0)),
        ],
        out_specs=(
            pl.BlockSpec((_BN, _H), lambda i: (i, 0)),
            pl.BlockSpec((_NET, _BN, _H), lambda i: (0, i, 0)),
        ),
        out_shape=(
            jax.ShapeDtypeStruct((_N, _H), jnp.float32),
            jax.ShapeDtypeStruct((_NET, _N, _H), jnp.float32),
        ),
    )(ap, h, wiht, whht, bih2, bhh2, wt, b3)


def _pool_body(h_ref, w_ref, b_ref, o_ref):
    pooled = jnp.mean(h_ref[...], axis=0, keepdims=True)
    o_ref[...] = (
        jnp.dot(pooled, w_ref[...], preferred_element_type=jnp.float32)
        + b_ref[...]
    )


def _pool_fc(h, fcwt, fcb2):
    return pl.pallas_call(
        _pool_body,
        out_shape=jax.ShapeDtypeStruct((1, _NCLS), jnp.float32),
    )(h, fcwt, fcb2)


@functools.cache
def _sc_message_pass_fn():
    mesh = plsc.VectorSubcoreMesh(core_axis_name="c", subcore_axis_name="s")

    @functools.partial(
        pl.kernel,
        mesh=mesh,
        out_type=jax.ShapeDtypeStruct((_NC, _NACC, _H), jnp.float32),
        scratch_types=[
            pltpu.VMEM((_K,), jnp.int32),
            pltpu.VMEM((_K,), jnp.int32),
            pltpu.VMEM((_K,), jnp.int32),
            pltpu.VMEM((_K,), jnp.int32),
            pltpu.VMEM((_K, _H), jnp.float32),
            pltpu.VMEM((_K, _H), jnp.float32),
            pltpu.VMEM_SHARED((_NACC, _H), jnp.float32),
            pltpu.VMEM_SHARED((_NS, _CHMAX * _K), jnp.int32),
            pltpu.SemaphoreType.DMA,
            pltpu.SemaphoreType.DMA,
            pltpu.SemaphoreType.DMA,
            pltpu.SemaphoreType.DMA,
        ],
    )
    def _sc_message_pass(t_hbm, gidx_hbm, didx_hbm, zeros_hbm, out_hbm,
                         gv, dv, gv1, dv1, rows, rows1, acc, idxsh,
                         semg0, semg1, sems0, sems1):
        c = lax.axis_index("c")
        s = lax.axis_index("s")
        wid = s * _NC + c
        pltpu.sync_copy(zeros_hbm.at[pl.ds(s * _ZROW, _ZROW)],
                        acc.at[pl.ds(s * _ZROW, _ZROW)])
        nch = jnp.where(c == 0, _CHA, _CHB)
        base = jnp.where(c == 0, s * _CHA, _NS * _CHA + s * _CHB) * _K
        pltpu.sync_copy(gidx_hbm.at[pl.ds(base, _CHMAX * _K)], idxsh.at[s])
        plsc.subcore_barrier()

        def chunk(j, gvb, dvb, rowsb, semg, sems, first):
            pltpu.sync_copy(idxsh.at[s, pl.ds(j * _K, _K)], gvb)
            if not first:
                pltpu.make_async_copy(rowsb, acc.at[dvb], sems).wait()
            cp = pltpu.async_copy(t_hbm.at[gvb], rowsb, semg)
            pltpu.sync_copy(didx_hbm.at[pl.ds(base + j * _K, _K)], dvb)
            cp.wait()
            pltpu.async_copy(rowsb, acc.at[dvb], sems, add=True)

        chunk(0, gv, dv, rows, semg0, sems0, True)
        chunk(1, gv1, dv1, rows1, semg1, sems1, True)

        def body(p, carry):
            chunk(2 * p, gv, dv, rows, semg0, sems0, False)
            chunk(2 * p + 1, gv1, dv1, rows1, semg1, sems1, False)
            return carry

        lax.fori_loop(1, nch // 2, body, 0)
        chunk(nch - 1, gv, dv, rows, semg0, sems0, False)
        pltpu.make_async_copy(rows, acc.at[dv], sems0).wait()
        pltpu.make_async_copy(rows1, acc.at[dv1], sems1).wait()
        plsc.subcore_barrier()
        pltpu.sync_copy(acc.at[pl.ds(s * _OROW, _OROW)],
                        out_hbm.at[c, pl.ds(s * _OROW, _OROW)])

    return _sc_message_pass


def kernel(node_features, edge_index, etypes, W0, b0, Wih0, Whh0, bih0, bhh0,
           W1, b1, Wih1, Whh1, bih1, bhh1, fcW, fcb):
    src = edge_index[0]
    dst = edge_index[1]
    gidx = etypes * _N + src
    pad = _EPAD - _E
    gidx = jnp.concatenate([gidx, jnp.zeros((pad,), jnp.int32)])
    didx = jnp.concatenate([dst, jnp.full((pad,), _N, jnp.int32)])
    zeros = jnp.zeros((_NACC, _H), jnp.float32)

    h = node_features
    layers = []
    for (W, b, Wih, Whh, bih, bhh) in (
        (W0, b0, Wih0, Whh0, bih0, bhh0),
        (W1, b1, Wih1, Whh1, bih1, bhh1),
    ):
        layers.append((jnp.transpose(W, (0, 2, 1)), b[:, None, :], Wih.T,
                       Whh.T, bih[None, :], bhh[None, :]))

    nsteps = 2 * _NSTEPS
    t = _etype_linear(h, layers[0][0], layers[0][1])
    for k in range(nsteps):
        wt, b3, wiht, whht, bih2, bhh2 = layers[k // _NSTEPS]
        ap = _sc_message_pass_fn()(t.reshape(_NET * _N, _H), gidx, didx, zeros)
        if k < nsteps - 1:
            nwt, nb3 = layers[(k + 1) // _NSTEPS][:2]
            h, t = _gru_fused(ap, h, wiht, whht, bih2, bhh2, nwt, nb3)
        else:
            h = _gru(ap, h, wiht, whht, bih2, bhh2)

    return _pool_fc(h, fcW.T, fcb[None, :])
